# baseline (speedup 1.0000x reference)
import jax
import jax.numpy as jnp
from jax.experimental import pallas as pl
from jax.experimental.pallas import tpu as pltpu

N_GRAPHS = 120
GRAPH_DIM = 25
DD = GRAPH_DIM * GRAPH_DIM
RATIO = 0.5
SE_ROWS = 128
TILE = 4096
CHUNK = 128


def _s_kernel(flat_t_ref, wq_ref, bq_ref, wk_ref, bk_ref, se_ref):
    flat_t = flat_t_ref[...]
    qt = jax.lax.dot_general(wq_ref[...], flat_t, (((1,), (0,)), ((), ())),
                             preferred_element_type=jnp.float32)
    qt = qt + bq_ref[...].T
    kt = jax.lax.dot_general(wk_ref[...], flat_t, (((1,), (0,)), ((), ())),
                             preferred_element_type=jnp.float32)
    kt = kt + bk_ref[...].T
    s = jax.lax.dot_general(qt, kt, (((0,), (0,)), ((), ())),
                            preferred_element_type=jnp.float32)
    rows = jax.lax.broadcasted_iota(jnp.int32, (SE_ROWS - N_GRAPHS, N_GRAPHS), 0)
    pad = jnp.where(rows == 0, 1.0, 0.0)
    se_ref[...] = jnp.concatenate([s, pad], axis=0)


def _fuse_kernel(logits_t_ref, se_ref, flat_t_ref, out_ref, attn_ref):
    for kc in range(TILE // CHUNK):
        sl = pl.ds(kc * CHUNK, CHUNK)
        lg = logits_t_ref[:, sl]
        mx = jnp.max(lg, axis=0, keepdims=True)
        iota = jax.lax.broadcasted_iota(jnp.int32, lg.shape, 0)
        rev = jnp.where(lg == mx, (N_GRAPHS - iota).astype(jnp.float32), 0.0)
        mrev = jnp.max(rev, axis=0, keepdims=True)
        onehotf = (rev == mrev).astype(jnp.float32)
        threshf = (lg > (RATIO * mx)).astype(jnp.float32)
        maskf = jnp.where(mx <= 0.0, onehotf, threshf)
        ext = jax.lax.dot_general(se_ref[...], maskf, (((1,), (0,)), ((), ())),
                                  preferred_element_type=jnp.float32)
        counts = ext[N_GRAPHS:N_GRAPHS + 1, :]
        row_mean = ext[:N_GRAPHS, :] / counts
        sel = maskf > 0.0
        m = jnp.max(jnp.where(sel, row_mean, -jnp.inf), axis=0, keepdims=True)
        p = jnp.where(sel, jnp.exp(row_mean - m), 0.0)
        attn_ref[:, sl] = p / jnp.sum(p, axis=0, keepdims=True)
    attn = attn_ref[...]
    for r in range(GRAPH_DIM):
        fr = flat_t_ref[pl.ds(r * GRAPH_DIM, GRAPH_DIM), :]
        out_ref[r, :, :] = jax.lax.dot_general(
            fr, attn, (((1,), (0,)), ((), ())),
            preferred_element_type=jnp.float32)


def kernel(logits, semantic_graphs, Wq, bq, Wk, bk):
    batch = logits.shape[0]
    logits_t = logits.T
    flat_t = semantic_graphs.reshape(N_GRAPHS, DD).T
    se = pl.pallas_call(
        _s_kernel,
        out_shape=jax.ShapeDtypeStruct((SE_ROWS, N_GRAPHS), jnp.float32),
    )(flat_t, Wq, bq.reshape(1, -1), Wk, bk.reshape(1, -1))
    out_t = pl.pallas_call(
        _fuse_kernel,
        grid=(batch // TILE,),
        in_specs=[
            pl.BlockSpec((N_GRAPHS, TILE), lambda i: (0, i)),
            pl.BlockSpec((SE_ROWS, N_GRAPHS), lambda i: (0, 0)),
            pl.BlockSpec((DD, N_GRAPHS), lambda i: (0, 0)),
        ],
        out_specs=pl.BlockSpec((GRAPH_DIM, GRAPH_DIM, TILE),
                               lambda i: (0, 0, i)),
        out_shape=jax.ShapeDtypeStruct((GRAPH_DIM, GRAPH_DIM, batch),
                                       jnp.float32),
        scratch_shapes=[pltpu.VMEM((N_GRAPHS, TILE), jnp.float32)],
    )(logits_t, se, flat_t)
    return out_t.transpose(2, 0, 1)


# speedup vs baseline: 1.0340x; 1.0340x over previous
import jax
import jax.numpy as jnp
from jax.experimental import pallas as pl
from jax.experimental.pallas import tpu as pltpu

N_GRAPHS = 120
GRAPH_DIM = 25
DD = GRAPH_DIM * GRAPH_DIM
RATIO = 0.5
SE_ROWS = 128
TILE = 2048
CHUNK = 256


def _s_kernel(flat_t_ref, wq_ref, bq_ref, wk_ref, bk_ref, se_ref):
    flat_t = flat_t_ref[...]
    qt = jax.lax.dot_general(wq_ref[...], flat_t, (((1,), (0,)), ((), ())),
                             preferred_element_type=jnp.float32)
    qt = qt + bq_ref[...].T
    kt = jax.lax.dot_general(wk_ref[...], flat_t, (((1,), (0,)), ((), ())),
                             preferred_element_type=jnp.float32)
    kt = kt + bk_ref[...].T
    s = jax.lax.dot_general(qt, kt, (((0,), (0,)), ((), ())),
                            preferred_element_type=jnp.float32)
    rows = jax.lax.broadcasted_iota(jnp.int32, (SE_ROWS - N_GRAPHS, N_GRAPHS), 0)
    pad = jnp.where(rows == 0, 1.0, 0.0)
    se_ref[...] = jnp.concatenate([s, pad], axis=0)


def _fuse_kernel(logits_t_ref, se_ref, flat_t_ref, out_ref, attn_ref):
    for kc in range(TILE // CHUNK):
        sl = pl.ds(kc * CHUNK, CHUNK)
        lg = logits_t_ref[:, sl]
        mx = jnp.max(lg, axis=0, keepdims=True)
        iota = jax.lax.broadcasted_iota(jnp.int32, lg.shape, 0)
        rev = jnp.where(lg == mx, (N_GRAPHS - iota).astype(jnp.float32), 0.0)
        mrev = jnp.max(rev, axis=0, keepdims=True)
        onehotf = (rev == mrev).astype(jnp.float32)
        threshf = (lg > (RATIO * mx)).astype(jnp.float32)
        maskf = jnp.where(mx <= 0.0, onehotf, threshf)
        ext = jax.lax.dot_general(se_ref[...], maskf, (((1,), (0,)), ((), ())),
                                  preferred_element_type=jnp.float32)
        counts = ext[N_GRAPHS:N_GRAPHS + 1, :]
        row_mean = ext[:N_GRAPHS, :] / counts
        sel = maskf > 0.0
        m = jnp.max(jnp.where(sel, row_mean, -jnp.inf), axis=0, keepdims=True)
        p = jnp.where(sel, jnp.exp(row_mean - m), 0.0)
        attn_ref[:, sl] = p / jnp.sum(p, axis=0, keepdims=True)
    attn = attn_ref[...]
    for r in range(GRAPH_DIM):
        fr = flat_t_ref[pl.ds(r * GRAPH_DIM, GRAPH_DIM), :]
        out_ref[r, :, :] = jax.lax.dot_general(
            fr, attn, (((1,), (0,)), ((), ())),
            preferred_element_type=jnp.float32)


def kernel(logits, semantic_graphs, Wq, bq, Wk, bk):
    batch = logits.shape[0]
    logits_t = logits.T
    flat_t = semantic_graphs.reshape(N_GRAPHS, DD).T
    se = pl.pallas_call(
        _s_kernel,
        out_shape=jax.ShapeDtypeStruct((SE_ROWS, N_GRAPHS), jnp.float32),
    )(flat_t, Wq, bq.reshape(1, -1), Wk, bk.reshape(1, -1))
    out_t = pl.pallas_call(
        _fuse_kernel,
        grid=(batch // TILE,),
        in_specs=[
            pl.BlockSpec((N_GRAPHS, TILE), lambda i: (0, i)),
            pl.BlockSpec((SE_ROWS, N_GRAPHS), lambda i: (0, 0)),
            pl.BlockSpec((DD, N_GRAPHS), lambda i: (0, 0)),
        ],
        out_specs=pl.BlockSpec((GRAPH_DIM, GRAPH_DIM, TILE),
                               lambda i: (0, 0, i)),
        out_shape=jax.ShapeDtypeStruct((GRAPH_DIM, GRAPH_DIM, batch),
                                       jnp.float32),
        scratch_shapes=[pltpu.VMEM((N_GRAPHS, TILE), jnp.float32)],
    )(logits_t, se, flat_t)
    return out_t.transpose(2, 0, 1)


# speedup vs baseline: 1.0438x; 1.0094x over previous
import jax
import jax.numpy as jnp
from jax.experimental import pallas as pl
from jax.experimental.pallas import tpu as pltpu

N_GRAPHS = 120
GRAPH_DIM = 25
DD = GRAPH_DIM * GRAPH_DIM
RATIO = 0.5
SE_ROWS = 128
TILE = 2048
CHUNK = 512


def _s_kernel(flat_t_ref, wq_ref, bq_ref, wk_ref, bk_ref, se_ref):
    flat_t = flat_t_ref[...]
    qt = jax.lax.dot_general(wq_ref[...], flat_t, (((1,), (0,)), ((), ())),
                             preferred_element_type=jnp.float32)
    qt = qt + bq_ref[...].T
    kt = jax.lax.dot_general(wk_ref[...], flat_t, (((1,), (0,)), ((), ())),
                             preferred_element_type=jnp.float32)
    kt = kt + bk_ref[...].T
    s = jax.lax.dot_general(qt, kt, (((0,), (0,)), ((), ())),
                            preferred_element_type=jnp.float32)
    rows = jax.lax.broadcasted_iota(jnp.int32, (SE_ROWS - N_GRAPHS, N_GRAPHS), 0)
    pad = jnp.where(rows == 0, 1.0, 0.0)
    se_ref[...] = jnp.concatenate([s, pad], axis=0)


def _fuse_kernel(logits_t_ref, se_ref, flat_t_ref, out_ref, attn_ref):
    for kc in range(TILE // CHUNK):
        sl = pl.ds(kc * CHUNK, CHUNK)
        lg = logits_t_ref[:, sl]
        mx = jnp.max(lg, axis=0, keepdims=True)
        iota = jax.lax.broadcasted_iota(jnp.int32, lg.shape, 0)
        rev = jnp.where(lg == mx, (N_GRAPHS - iota).astype(jnp.float32), 0.0)
        mrev = jnp.max(rev, axis=0, keepdims=True)
        onehotf = (rev == mrev).astype(jnp.float32)
        threshf = (lg > (RATIO * mx)).astype(jnp.float32)
        maskf = jnp.where(mx <= 0.0, onehotf, threshf)
        ext = jax.lax.dot_general(se_ref[...], maskf, (((1,), (0,)), ((), ())),
                                  preferred_element_type=jnp.float32)
        counts = ext[N_GRAPHS:N_GRAPHS + 1, :]
        row_mean = ext[:N_GRAPHS, :] / counts
        sel = maskf > 0.0
        m = jnp.max(jnp.where(sel, row_mean, -jnp.inf), axis=0, keepdims=True)
        p = jnp.where(sel, jnp.exp(row_mean - m), 0.0)
        attn_ref[:, sl] = p / jnp.sum(p, axis=0, keepdims=True)
    attn = attn_ref[...]
    for r in range(GRAPH_DIM):
        fr = flat_t_ref[pl.ds(r * GRAPH_DIM, GRAPH_DIM), :]
        out_ref[r, :, :] = jax.lax.dot_general(
            fr, attn, (((1,), (0,)), ((), ())),
            preferred_element_type=jnp.float32)


def kernel(logits, semantic_graphs, Wq, bq, Wk, bk):
    batch = logits.shape[0]
    logits_t = logits.T
    flat_t = semantic_graphs.reshape(N_GRAPHS, DD).T
    se = pl.pallas_call(
        _s_kernel,
        out_shape=jax.ShapeDtypeStruct((SE_ROWS, N_GRAPHS), jnp.float32),
    )(flat_t, Wq, bq.reshape(1, -1), Wk, bk.reshape(1, -1))
    out_t = pl.pallas_call(
        _fuse_kernel,
        grid=(batch // TILE,),
        in_specs=[
            pl.BlockSpec((N_GRAPHS, TILE), lambda i: (0, i)),
            pl.BlockSpec((SE_ROWS, N_GRAPHS), lambda i: (0, 0)),
            pl.BlockSpec((DD, N_GRAPHS), lambda i: (0, 0)),
        ],
        out_specs=pl.BlockSpec((GRAPH_DIM, GRAPH_DIM, TILE),
                               lambda i: (0, 0, i)),
        out_shape=jax.ShapeDtypeStruct((GRAPH_DIM, GRAPH_DIM, batch),
                                       jnp.float32),
        scratch_shapes=[pltpu.VMEM((N_GRAPHS, TILE), jnp.float32)],
    )(logits_t, se, flat_t)
    return out_t.transpose(2, 0, 1)
